# baseline (speedup 1.0000x reference)
import functools

import jax
import jax.numpy as jnp
from jax import lax
from jax.experimental import pallas as pl
from jax.experimental.pallas import tpu as pltpu
from jax.experimental.pallas import tpu_sc as plsc

_CODEBOOK_K = 8192
_CODE_OFFSET = 40000
_ID_BOV = 49000
_ID_BOA = 49002
_N_CODES = 32
_WIN = 40


def _sc_gather(ids, table):
  bsz, l_seq = ids.shape
  n = bsz * l_seq
  d = table.shape[1]
  info = plsc.get_sparse_core_info()
  nw = info.num_cores * info.num_subcores
  per_w = n // nw
  w_per_b = l_seq // per_w
  ch = 8
  n_ch = per_w // ch
  mesh = plsc.VectorSubcoreMesh(core_axis_name="c", subcore_axis_name="s")

  @functools.partial(
      pl.kernel,
      mesh=mesh,
      out_type=jax.ShapeDtypeStruct((bsz, l_seq, d), jnp.float32),
      scratch_types=[
          pltpu.VMEM((per_w,), jnp.int32),
          pltpu.VMEM((ch, d), jnp.float32),
          pltpu.VMEM((ch, d), jnp.float32),
          pltpu.VMEM((ch, d), jnp.float32),
          pltpu.SemaphoreType.DMA,
          pltpu.SemaphoreType.DMA,
          pltpu.SemaphoreType.DMA,
          pltpu.SemaphoreType.DMA,
          pltpu.SemaphoreType.DMA,
          pltpu.SemaphoreType.DMA,
      ],
  )
  def gather_kernel(ids_hbm, table_hbm, out_hbm, idx_v, rows0, rows1, rows2,
                    sg0, sg1, sg2, ss0, ss1, ss2):
    wid = lax.axis_index("s") * info.num_cores + lax.axis_index("c")
    b_idx = wid // w_per_b
    col = (wid % w_per_b) * per_w
    pltpu.sync_copy(ids_hbm.at[b_idx, pl.ds(col, per_w)], idx_v)
    bufs = (rows0, rows1, rows2)
    sgs = (sg0, sg1, sg2)
    sss = (ss0, ss1, ss2)

    def g_start(c, b):
      pltpu.async_copy(table_hbm.at[idx_v.at[pl.ds(c * ch, ch)]], bufs[b],
                       sgs[b])

    def g_wait(b):
      pltpu.make_async_copy(table_hbm.at[idx_v.at[pl.ds(0, ch)]], bufs[b],
                            sgs[b]).wait()

    def s_start(c, b):
      pltpu.async_copy(bufs[b], out_hbm.at[b_idx, pl.ds(col + c * ch, ch), :],
                       sss[b])

    def s_wait(c, b):
      pltpu.make_async_copy(bufs[b],
                            out_hbm.at[b_idx, pl.ds(col + c * ch, ch), :],
                            sss[b]).wait()

    def step(c, b, swait_prev, gstart_next):
      g_wait(b)
      s_start(c, b)
      if swait_prev:
        s_wait(c - 1, (b + 2) % 3)
      if gstart_next:
        g_start(c + 2, (b + 2) % 3)

    g_start(0, 0)
    g_start(1, 1)
    step(0, 0, False, True)

    def body(i, carry):
      c0 = 3 * i + 1
      step(c0, 1, True, True)
      step(c0 + 1, 2, True, True)
      step(c0 + 2, 0, True, True)
      return carry

    lax.fori_loop(0, (n_ch - 5) // 3, body, 0)
    for c in range(n_ch - 4, n_ch):
      step(c, c % 3, True, c + 2 < n_ch)
    s_wait(n_ch - 1, (n_ch - 1) % 3)

  return gather_kernel(ids, table)


def _proj_kernel(ids3_ref, cb_ref, w_ref, b_ref, shifted_ref, pos_ref):
  b_batches, sub, lane = ids3_ref.shape
  l_seq = sub * lane
  flat_pos = (lax.broadcasted_iota(jnp.int32, (sub, lane), 0) * lane
              + lax.broadcasted_iota(jnp.int32, (sub, lane), 1))
  for b in range(b_batches):
    row = ids3_ref[b]
    p_bov = jnp.min(jnp.where(row == _ID_BOV, flat_pos, l_seq))
    p_boa = jnp.min(jnp.where(row == _ID_BOA, flat_pos, l_seq))
    lane_iota = lax.broadcasted_iota(jnp.int32, (1, 128), 1)
    pos_ref[pl.ds(b, 1), :] = jnp.where(lane_iota == 0, p_bov,
                                        jnp.where(lane_iota == 1, p_boa, 0))
    jg = lax.broadcasted_iota(jnp.int32, (2 * _N_CODES, 1), 0)
    t = jnp.where(jg < _N_CODES, p_bov + 1 + jg, p_boa + 1 + jg - _N_CODES)
    rmask = (lax.broadcasted_iota(jnp.int32, (2 * _N_CODES, sub), 1)
             == t // lane).astype(jnp.float32)
    cmask = (lax.broadcasted_iota(jnp.int32, (2 * _N_CODES, lane), 1)
             == t % lane).astype(jnp.float32)
    row_f = row.astype(jnp.float32)
    picked = jnp.dot(rmask, row_f, preferred_element_type=jnp.float32,
                     precision=lax.Precision.HIGHEST)
    codes = jnp.sum(picked * cmask, axis=1, keepdims=True).astype(jnp.int32)
    codes = codes - _CODE_OFFSET
    onehot = (lax.broadcasted_iota(jnp.int32, (2 * _N_CODES, _CODEBOOK_K), 1)
              == codes).astype(jnp.float32)
    emb = jnp.dot(onehot, cb_ref[...], preferred_element_type=jnp.float32,
                  precision=lax.Precision.HIGHEST)
    proj = (jnp.dot(emb, w_ref[...], preferred_element_type=jnp.float32,
                    precision=lax.Precision.HIGHEST) + b_ref[...])
    rowi = lax.broadcasted_iota(jnp.int32, (_WIN, _N_CODES), 0)
    colj = lax.broadcasted_iota(jnp.int32, (_WIN, _N_CODES), 1)
    for g, start in enumerate((p_bov + 1, p_boa + 1)):
      o = start % 8
      perm = (colj == rowi - o).astype(jnp.float32)
      shifted_ref[b, g] = jnp.dot(
          perm, proj[g * _N_CODES:(g + 1) * _N_CODES],
          preferred_element_type=jnp.float32,
          precision=lax.Precision.HIGHEST)


def _tc_proj(input_ids, codebook, w_proj, b_proj):
  bsz, l_seq = input_ids.shape
  d = w_proj.shape[1]
  ids3 = input_ids.reshape(bsz, l_seq // 128, 128)
  return pl.pallas_call(
      _proj_kernel,
      out_shape=(
          jax.ShapeDtypeStruct((bsz, 2, _WIN, d), jnp.float32),
          jax.ShapeDtypeStruct((bsz, 128), jnp.int32),
      ),
      in_specs=[
          pl.BlockSpec(memory_space=pltpu.VMEM),
          pl.BlockSpec(memory_space=pltpu.VMEM),
          pl.BlockSpec(memory_space=pltpu.VMEM),
          pl.BlockSpec(memory_space=pltpu.VMEM),
      ],
      out_specs=(
          pl.BlockSpec(memory_space=pltpu.VMEM),
          pl.BlockSpec(memory_space=pltpu.VMEM),
      ),
  )(ids3, codebook, w_proj, b_proj.reshape(1, d))


def _merge_kernel(pos_ref, sh_ref, out0_ref, out_ref, ebuf_v, sbuf_v,
                  sem, sem2, sem3):
  del out0_ref
  b_batches = sh_ref.shape[0]
  n_win = 2 * b_batches
  iota8 = lax.broadcasted_iota(jnp.int32, (8, 1), 0)

  def win_params(k):
    b, g = k // 2, k % 2
    start = pos_ref[b, g] + 1
    a = pl.multiple_of((start // 8) * 8, 8)
    return b, g, a, start - a

  interiors = []
  for k in range(n_win):
    b, g, a, o = win_params(k)
    cp = pltpu.make_async_copy(sh_ref.at[b, g, pl.ds(8, 24), :],
                               out_ref.at[b, pl.ds(a + 8, 24), :], sem3)
    cp.start()
    interiors.append(cp)
  reads = []
  for k in range(n_win):
    b, g, a, o = win_params(k)
    for e, off in enumerate((0, _WIN - 8)):
      cp = pltpu.make_async_copy(out_ref.at[b, pl.ds(a + off, 8), :],
                                 ebuf_v.at[2 * k + e], sem)
      cp.start()
      cp2 = pltpu.make_async_copy(sh_ref.at[b, g, pl.ds(off, 8), :],
                                  sbuf_v.at[2 * k + e], sem2)
      cp2.start()
      reads.append((cp, cp2))
  for cp, cp2 in reads:
    cp.wait()
    cp2.wait()
  writes = []
  for k in range(n_win):
    b, g, a, o = win_params(k)
    for e, off in enumerate((0, _WIN - 8)):
      patched = (iota8 >= o) if e == 0 else (iota8 < o)
      ebuf_v[2 * k + e] = jnp.where(patched, sbuf_v[2 * k + e],
                                    ebuf_v[2 * k + e])
      cp = pltpu.make_async_copy(ebuf_v.at[2 * k + e],
                                 out_ref.at[b, pl.ds(a + off, 8), :], sem)
      cp.start()
      writes.append(cp)
  for cp in writes:
    cp.wait()
  for cp in interiors:
    cp.wait()


def _tc_merge(out, pos, shifted):
  bsz, l_seq, d = out.shape
  return pl.pallas_call(
      _merge_kernel,
      out_shape=jax.ShapeDtypeStruct((bsz, l_seq, d), jnp.float32),
      in_specs=[
          pl.BlockSpec(memory_space=pltpu.SMEM),
          pl.BlockSpec(memory_space=pl.ANY),
          pl.BlockSpec(memory_space=pl.ANY),
      ],
      out_specs=pl.BlockSpec(memory_space=pl.ANY),
      scratch_shapes=[
          pltpu.VMEM((4 * bsz, 8, d), jnp.float32),
          pltpu.VMEM((4 * bsz, 8, d), jnp.float32),
          pltpu.SemaphoreType.DMA,
          pltpu.SemaphoreType.DMA,
          pltpu.SemaphoreType.DMA,
      ],
      input_output_aliases={2: 0},
  )(pos, shifted, out)


def kernel(input_ids, text_table, codebook, W_proj, b_proj):
  shifted, pos = _tc_proj(input_ids, codebook, W_proj, b_proj)
  out = _sc_gather(input_ids, text_table)
  return _tc_merge(out, pos, shifted)


# speedup vs baseline: 1.7194x; 1.7194x over previous
import functools

import jax
import jax.numpy as jnp
from jax import lax
from jax.experimental import pallas as pl
from jax.experimental.pallas import tpu as pltpu
from jax.experimental.pallas import tpu_sc as plsc

_CODEBOOK_K = 8192
_CODE_OFFSET = 40000
_ID_BOV = 49000
_ID_BOA = 49002
_N_CODES = 32
_WIN = 40


def _sc_gather(ids, table):
  bsz, l_seq = ids.shape
  n = bsz * l_seq
  d = table.shape[1]
  info = plsc.get_sparse_core_info()
  nw = info.num_cores * info.num_subcores
  per_w = n // nw
  w_per_b = l_seq // per_w
  ch = 8
  n_ch = per_w // ch
  mesh = plsc.VectorSubcoreMesh(core_axis_name="c", subcore_axis_name="s")

  @functools.partial(
      pl.kernel,
      mesh=mesh,
      out_type=jax.ShapeDtypeStruct((bsz, l_seq, d), jnp.float32),
      scratch_types=[
          pltpu.VMEM((per_w,), jnp.int32),
          pltpu.VMEM((ch, d), jnp.float32),
          pltpu.VMEM((ch, d), jnp.float32),
          pltpu.VMEM((ch, d), jnp.float32),
          pltpu.SemaphoreType.DMA,
          pltpu.SemaphoreType.DMA,
          pltpu.SemaphoreType.DMA,
          pltpu.SemaphoreType.DMA,
          pltpu.SemaphoreType.DMA,
          pltpu.SemaphoreType.DMA,
      ],
  )
  def gather_kernel(ids_hbm, table_hbm, out_hbm, idx_v, rows0, rows1, rows2,
                    sg0, sg1, sg2, ss0, ss1, ss2):
    wid = lax.axis_index("s") * info.num_cores + lax.axis_index("c")
    b_idx = wid // w_per_b
    col = (wid % w_per_b) * per_w
    pltpu.sync_copy(ids_hbm.at[b_idx, pl.ds(col, per_w)], idx_v)
    bufs = (rows0, rows1, rows2)
    sgs = (sg0, sg1, sg2)
    sss = (ss0, ss1, ss2)

    def g_start(c, b):
      pltpu.async_copy(table_hbm.at[idx_v.at[pl.ds(c * ch, ch)]], bufs[b],
                       sgs[b])

    def g_wait(b):
      pltpu.make_async_copy(table_hbm.at[idx_v.at[pl.ds(0, ch)]], bufs[b],
                            sgs[b]).wait()

    def s_start(c, b):
      pltpu.async_copy(bufs[b], out_hbm.at[b_idx, pl.ds(col + c * ch, ch), :],
                       sss[b])

    def s_wait(c, b):
      pltpu.make_async_copy(bufs[b],
                            out_hbm.at[b_idx, pl.ds(col + c * ch, ch), :],
                            sss[b]).wait()

    def step(c, b, swait_prev, gstart_next):
      g_wait(b)
      s_start(c, b)
      if swait_prev:
        s_wait(c - 1, (b + 2) % 3)
      if gstart_next:
        g_start(c + 2, (b + 2) % 3)

    g_start(0, 0)
    g_start(1, 1)
    step(0, 0, False, True)

    def body(i, carry):
      c0 = 3 * i + 1
      step(c0, 1, True, True)
      step(c0 + 1, 2, True, True)
      step(c0 + 2, 0, True, True)
      return carry

    lax.fori_loop(0, (n_ch - 5) // 3, body, 0)
    for c in range(n_ch - 4, n_ch):
      step(c, c % 3, True, c + 2 < n_ch)
    s_wait(n_ch - 1, (n_ch - 1) % 3)

  return gather_kernel(ids, table)


def _proj_kernel(ids3_ref, cb_ref, w_ref, b_ref, shifted_ref, pos_ref):
  b_batches, sub, lane = ids3_ref.shape
  l_seq = sub * lane
  flat_pos = (lax.broadcasted_iota(jnp.int32, (sub, lane), 0) * lane
              + lax.broadcasted_iota(jnp.int32, (sub, lane), 1))
  for b in range(b_batches):
    row = ids3_ref[b]
    p_bov = jnp.min(jnp.where(row == _ID_BOV, flat_pos, l_seq))
    p_boa = jnp.min(jnp.where(row == _ID_BOA, flat_pos, l_seq))
    lane_iota = lax.broadcasted_iota(jnp.int32, (1, 128), 1)
    pos_ref[pl.ds(b, 1), :] = jnp.where(lane_iota == 0, p_bov,
                                        jnp.where(lane_iota == 1, p_boa, 0))
    jg = lax.broadcasted_iota(jnp.int32, (2 * _N_CODES, 1), 0)
    t = jnp.where(jg < _N_CODES, p_bov + 1 + jg, p_boa + 1 + jg - _N_CODES)
    rmask = (lax.broadcasted_iota(jnp.int32, (2 * _N_CODES, sub), 1)
             == t // lane).astype(jnp.float32)
    cmask = (lax.broadcasted_iota(jnp.int32, (2 * _N_CODES, lane), 1)
             == t % lane).astype(jnp.float32)
    row_f = row.astype(jnp.float32)
    picked = jnp.dot(rmask, row_f, preferred_element_type=jnp.float32,
                     precision=lax.Precision.HIGHEST)
    codes = jnp.sum(picked * cmask, axis=1, keepdims=True).astype(jnp.int32)
    codes = codes - _CODE_OFFSET
    onehot = (lax.broadcasted_iota(jnp.int32, (2 * _N_CODES, _CODEBOOK_K), 1)
              == codes).astype(jnp.float32)
    emb = jnp.dot(onehot, cb_ref[...], preferred_element_type=jnp.float32,
                  precision=lax.Precision.HIGHEST)
    proj = (jnp.dot(emb, w_ref[...], preferred_element_type=jnp.float32,
                    precision=lax.Precision.HIGHEST) + b_ref[...])
    rowi = lax.broadcasted_iota(jnp.int32, (_WIN, _N_CODES), 0)
    colj = lax.broadcasted_iota(jnp.int32, (_WIN, _N_CODES), 1)
    for g, start in enumerate((p_bov + 1, p_boa + 1)):
      o = start % 8
      perm = (colj == rowi - o).astype(jnp.float32)
      shifted_ref[b, g] = jnp.dot(
          perm, proj[g * _N_CODES:(g + 1) * _N_CODES],
          preferred_element_type=jnp.float32,
          precision=lax.Precision.HIGHEST)


def _tc_proj(input_ids, codebook, w_proj, b_proj):
  bsz, l_seq = input_ids.shape
  d = w_proj.shape[1]
  ids3 = input_ids.reshape(bsz, l_seq // 128, 128)
  return pl.pallas_call(
      _proj_kernel,
      out_shape=(
          jax.ShapeDtypeStruct((bsz, 2, _WIN, d), jnp.float32),
          jax.ShapeDtypeStruct((bsz, 128), jnp.int32),
      ),
      in_specs=[
          pl.BlockSpec(memory_space=pltpu.VMEM),
          pl.BlockSpec(memory_space=pltpu.VMEM),
          pl.BlockSpec(memory_space=pltpu.VMEM),
          pl.BlockSpec(memory_space=pltpu.VMEM),
      ],
      out_specs=(
          pl.BlockSpec(memory_space=pltpu.VMEM),
          pl.BlockSpec(memory_space=pltpu.VMEM),
      ),
  )(ids3, codebook, w_proj, b_proj.reshape(1, d))


def _merge_kernel(pos_ref, sh_ref, out0_ref, out_ref, buf_v, sem, sem2):
  del out0_ref
  b_batches = sh_ref.shape[0]
  n_win = 2 * b_batches
  rowi = lax.broadcasted_iota(jnp.int32, (_WIN, 1), 0)

  def win_params(k):
    b, g = k // 2, k % 2
    start = pos_ref[b, g] + 1
    a = pl.multiple_of((start // 8) * 8, 8)
    return b, g, a, start - a

  reads = []
  for k in range(n_win):
    b, g, a, o = win_params(k)
    cp = pltpu.make_async_copy(out_ref.at[b, pl.ds(a, _WIN), :],
                               buf_v.at[k], sem)
    cp.start()
    reads.append(cp)
  for cp in reads:
    cp.wait()
  writes = []
  for k in range(n_win):
    b, g, a, o = win_params(k)
    sel = (rowi >= o) & (rowi < o + _N_CODES)
    buf_v[k] = jnp.where(sel, sh_ref[b, g], buf_v[k])
    cp = pltpu.make_async_copy(buf_v.at[k], out_ref.at[b, pl.ds(a, _WIN), :],
                               sem2)
    cp.start()
    writes.append(cp)
  for cp in writes:
    cp.wait()


def _tc_merge(out, pos, shifted):
  bsz, l_seq, d = out.shape
  return pl.pallas_call(
      _merge_kernel,
      out_shape=jax.ShapeDtypeStruct((bsz, l_seq, d), jnp.float32),
      in_specs=[
          pl.BlockSpec(memory_space=pltpu.SMEM),
          pl.BlockSpec(memory_space=pltpu.VMEM),
          pl.BlockSpec(memory_space=pl.ANY),
      ],
      out_specs=pl.BlockSpec(memory_space=pl.ANY),
      scratch_shapes=[
          pltpu.VMEM((2 * bsz, _WIN, d), jnp.float32),
          pltpu.SemaphoreType.DMA,
          pltpu.SemaphoreType.DMA,
      ],
      input_output_aliases={2: 0},
  )(pos, shifted, out)


def kernel(input_ids, text_table, codebook, W_proj, b_proj):
  shifted, pos = _tc_proj(input_ids, codebook, W_proj, b_proj)
  out = _sc_gather(input_ids, text_table)
  return _tc_merge(out, pos, shifted)
